# baseline (speedup 1.0000x reference)
import functools

import jax
import jax.numpy as jnp
from jax import lax
from jax.experimental import pallas as pl
from jax.experimental.pallas import tpu as pltpu
from jax.experimental.pallas import tpu_sc as plsc

_K = 65536
_NW = 32
_CHUNK = _K // _NW
_SUB = 512
_J = _SUB // 16

_LN2 = 0.6931471805599453
_C1 = 2.885390081777927
_C3 = 0.961796693925976
_C5 = 0.577078016355585
_C7 = 0.412198583111132


def _log2(x):
    bits = lax.bitcast_convert_type(x, jnp.int32)
    e = ((bits >> 23) & 0xFF) - 127
    m = lax.bitcast_convert_type((bits & 0x007FFFFF) | 0x3F800000,
                                 jnp.float32)
    s = (m - 1.0) / (m + 1.0)
    u = s * s
    p = s * (_C1 + u * (_C3 + u * (_C5 + u * _C7)))
    return e.astype(jnp.float32) + p


def _sc_body(attx_hbm, attg_hbm, spap_hbm, spat_hbm, conp_hbm, cont_hbm,
             out_hbm, attx_v, attg_v, spap_v, spat_v, conp_v, cont_v, out_v):
    wid = lax.axis_index("s") * 2 + lax.axis_index("c")
    base = wid * _CHUNK

    z = jnp.zeros((16,), jnp.float32)
    acc = (z, z, z)
    for k in range(_CHUNK // _SUB):
        off = base + k * _SUB
        pltpu.sync_copy(attx_hbm.at[:, pl.ds(off, _SUB)], attx_v)
        pltpu.sync_copy(attg_hbm.at[pl.ds(off, _SUB)], attg_v)
        pltpu.sync_copy(spap_hbm.at[:, pl.ds(off, _SUB)], spap_v)
        pltpu.sync_copy(spat_hbm.at[:, pl.ds(off, _SUB)], spat_v)
        pltpu.sync_copy(conp_hbm.at[:, pl.ds(off, _SUB)], conp_v)
        pltpu.sync_copy(cont_hbm.at[:, pl.ds(off, _SUB)], cont_v)
        acc = _subchunk(attx_v, attg_v, spap_v, spat_v, conp_v, cont_v, acc)
    ce_a, spa_a, con_a = acc

    out_v[pl.ds(0, 16)] = ce_a
    out_v[pl.ds(16, 16)] = spa_a
    out_v[pl.ds(32, 16)] = con_a
    pltpu.sync_copy(out_v, out_hbm.at[wid])


def _subchunk(attx_v, attg_v, spap_v, spat_v, conp_v, cont_v, carry0):
    def step(j, carry):
        ce_a, spa_a, con_a = carry
        sl = pl.ds(j * 16, 16)

        x0 = attx_v[0, sl]
        x1 = attx_v[1, sl]
        x2 = attx_v[2, sl]
        g = attg_v[sl]
        m = jnp.maximum(jnp.maximum(x0, x1), x2)
        ssum = jnp.exp(x0 - m) + jnp.exp(x1 - m) + jnp.exp(x2 - m)
        lse = m + _log2(ssum) * _LN2
        xl = jnp.where(g == 0, x0, jnp.where(g == 1, x1, x2))
        ce_a = ce_a + (lse - xl)

        for c in range(6):
            q = jnp.abs(spap_v[c, sl] + spat_v[c, sl].astype(jnp.float32)
                        - 1.0)
            spa_a = spa_a + _log2(jnp.maximum(q, 1e-7))
        for c in range(17):
            q = jnp.abs(conp_v[c, sl] + cont_v[c, sl].astype(jnp.float32)
                        - 1.0)
            con_a = con_a + _log2(jnp.maximum(q, 1e-7))
        return ce_a, spa_a, con_a

    return lax.fori_loop(0, _J, step, carry0)


def kernel(att_logits, spa_probs, con_probs, att_gt, spa_gt, con_gt):
    attx = att_logits.T
    attg = att_gt.astype(jnp.int32)
    spap = spa_probs.T
    spat = spa_gt.T
    conp = con_probs.T
    cont = con_gt.T

    mesh = plsc.VectorSubcoreMesh(core_axis_name="c", subcore_axis_name="s")
    run = functools.partial(
        pl.kernel,
        mesh=mesh,
        out_type=jax.ShapeDtypeStruct((_NW, 48), jnp.float32),
        scratch_types=[
            pltpu.VMEM((3, _SUB), jnp.float32),
            pltpu.VMEM((_SUB,), jnp.int32),
            pltpu.VMEM((6, _SUB), jnp.float32),
            pltpu.VMEM((6, _SUB), jnp.int32),
            pltpu.VMEM((17, _SUB), jnp.float32),
            pltpu.VMEM((17, _SUB), jnp.int32),
            pltpu.VMEM((48,), jnp.float32),
        ],
    )(_sc_body)

    parts = run(attx, attg, spap, spat, conp, cont)
    att_loss = jnp.sum(parts[:, 0:16]) * (1.0 / _K)
    spa_loss = jnp.sum(parts[:, 16:32]) * (-_LN2 / (_K * 6))
    con_loss = jnp.sum(parts[:, 32:48]) * (-_LN2 / (_K * 17))
    total = att_loss + spa_loss + con_loss
    return (att_loss, spa_loss, con_loss, total)


# speedup vs baseline: 6.5645x; 6.5645x over previous
import jax
import jax.numpy as jnp
from jax.experimental import pallas as pl
from jax.experimental.pallas import tpu as pltpu

_K = 65536
_GRID = 4
_B = _K // _GRID
_AB = _K // 128 // _GRID

_LN2 = 0.6931471805599453


def _loss_kernel(attx_ref, attg_ref, spap_ref, spat_ref, conp_ref, cont_ref,
                 out_ref, ce_acc, spa_acc, con_acc):
    i = pl.program_id(0)

    @pl.when(i == 0)
    def _init():
        ce_acc[...] = jnp.zeros_like(ce_acc)
        spa_acc[...] = jnp.zeros_like(spa_acc)
        con_acc[...] = jnp.zeros_like(con_acc)

    x0 = attx_ref[0]
    x1 = attx_ref[1]
    x2 = attx_ref[2]
    g = attg_ref[...]
    m = jnp.maximum(jnp.maximum(x0, x1), x2)
    s = jnp.exp(x0 - m) + jnp.exp(x1 - m) + jnp.exp(x2 - m)
    lse = m + jnp.log(s)
    xl = jnp.where(g == 0, x0, jnp.where(g == 1, x1, x2))
    ce_acc[...] += lse - xl

    qs = jnp.abs(spap_ref[...] + spat_ref[...].astype(jnp.float32) - 1.0)
    spa_acc[...] += jnp.log2(jnp.maximum(qs, 1e-7))

    qc = jnp.abs(conp_ref[...] + cont_ref[...].astype(jnp.float32) - 1.0)
    con_acc[...] += jnp.log2(jnp.maximum(qc, 1e-7))

    @pl.when(i == _GRID - 1)
    def _fin():
        att = jnp.sum(ce_acc[...]) * (1.0 / _K)
        spa = jnp.sum(spa_acc[...]) * (-_LN2 / (_K * 6))
        con = jnp.sum(con_acc[...]) * (-_LN2 / (_K * 17))
        out_ref[0] = att
        out_ref[1] = spa
        out_ref[2] = con
        out_ref[3] = att + spa + con


def kernel(att_logits, spa_probs, con_probs, att_gt, spa_gt, con_gt):
    attx = att_logits.T.reshape(3, _K // 128, 128)
    attg = att_gt.astype(jnp.int32).reshape(_K // 128, 128)
    spap = spa_probs.T
    spat = spa_gt.T
    conp = con_probs.T
    cont = con_gt.T

    out = pl.pallas_call(
        _loss_kernel,
        grid=(_GRID,),
        in_specs=[
            pl.BlockSpec((3, _AB, 128), lambda i: (0, i, 0)),
            pl.BlockSpec((_AB, 128), lambda i: (i, 0)),
            pl.BlockSpec((6, _B), lambda i: (0, i)),
            pl.BlockSpec((6, _B), lambda i: (0, i)),
            pl.BlockSpec((17, _B), lambda i: (0, i)),
            pl.BlockSpec((17, _B), lambda i: (0, i)),
        ],
        out_specs=pl.BlockSpec(memory_space=pltpu.MemorySpace.SMEM),
        out_shape=jax.ShapeDtypeStruct((4,), jnp.float32),
        scratch_shapes=[
            pltpu.VMEM((_AB, 128), jnp.float32),
            pltpu.VMEM((6, _B), jnp.float32),
            pltpu.VMEM((17, _B), jnp.float32),
        ],
        compiler_params=pltpu.CompilerParams(
            dimension_semantics=("arbitrary",),
        ),
    )(attx, attg, spap, spat, conp, cont)

    return (out[0], out[1], out[2], out[3])


# speedup vs baseline: 6.6507x; 1.0131x over previous
import jax
import jax.numpy as jnp
from jax.experimental import pallas as pl
from jax.experimental.pallas import tpu as pltpu

_K = 65536
_GRID = 4
_B = _K // _GRID
_AB = _K // 128 // _GRID

_LN2 = 0.6931471805599453


def _loss_kernel(attx_ref, attg_ref, spap_ref, spat_ref, conp_ref, cont_ref,
                 out_ref, ce_acc, spa_acc, con_acc):
    i = pl.program_id(0)

    @pl.when(i == 0)
    def _init():
        ce_acc[...] = jnp.zeros_like(ce_acc)
        spa_acc[...] = jnp.zeros_like(spa_acc)
        con_acc[...] = jnp.zeros_like(con_acc)

    x0 = attx_ref[0]
    x1 = attx_ref[1]
    x2 = attx_ref[2]
    g = attg_ref[...]
    m = jnp.maximum(jnp.maximum(x0, x1), x2)
    s = jnp.exp(x0 - m) + jnp.exp(x1 - m) + jnp.exp(x2 - m)
    lse = m + jnp.log(s)
    xl = jnp.where(g == 0, x0, jnp.where(g == 1, x1, x2))
    ce_acc[...] += lse - xl

    qs = jnp.abs(spap_ref[...] + spat_ref[...].astype(jnp.float32) - 1.0)
    spa_acc[...] += jnp.log2(jnp.maximum(qs, 1e-7))

    qc = jnp.abs(conp_ref[...] + cont_ref[...].astype(jnp.float32) - 1.0)
    lc = jnp.log2(jnp.maximum(qc, 1e-7))
    con_acc[...] += jnp.sum(lc, axis=0, keepdims=True)

    @pl.when(i == _GRID - 1)
    def _fin():
        att = jnp.sum(ce_acc[...]) * (1.0 / _K)
        spa = jnp.sum(spa_acc[...]) * (-_LN2 / (_K * 6))
        con = jnp.sum(con_acc[...]) * (-_LN2 / (_K * 17))
        out_ref[0] = att
        out_ref[1] = spa
        out_ref[2] = con
        out_ref[3] = att + spa + con


def kernel(att_logits, spa_probs, con_probs, att_gt, spa_gt, con_gt):
    attx = att_logits.T.reshape(3, _K // 128, 128)
    attg = att_gt.astype(jnp.int32).reshape(_K // 128, 128)
    spap = spa_probs.T
    spat = spa_gt.T
    conp = con_probs.T
    cont = con_gt.T

    out = pl.pallas_call(
        _loss_kernel,
        grid=(_GRID,),
        in_specs=[
            pl.BlockSpec((3, _AB, 128), lambda i: (0, i, 0)),
            pl.BlockSpec((_AB, 128), lambda i: (i, 0)),
            pl.BlockSpec((6, _B), lambda i: (0, i)),
            pl.BlockSpec((6, _B), lambda i: (0, i)),
            pl.BlockSpec((17, _B), lambda i: (0, i)),
            pl.BlockSpec((17, _B), lambda i: (0, i)),
        ],
        out_specs=pl.BlockSpec(memory_space=pltpu.MemorySpace.SMEM),
        out_shape=jax.ShapeDtypeStruct((4,), jnp.float32),
        scratch_shapes=[
            pltpu.VMEM((_AB, 128), jnp.float32),
            pltpu.VMEM((6, _B), jnp.float32),
            pltpu.VMEM((1, _B), jnp.float32),
        ],
        compiler_params=pltpu.CompilerParams(
            dimension_semantics=("arbitrary",),
        ),
    )(attx, attg, spap, spat, conp, cont)

    return (out[0], out[1], out[2], out[3])


# speedup vs baseline: 6.7472x; 1.0145x over previous
import jax
import jax.numpy as jnp
from jax.experimental import pallas as pl
from jax.experimental.pallas import tpu as pltpu

_K = 65536
_GRID = 4
_B = _K // _GRID
_AB = _K // 128 // _GRID

_LN2 = 0.6931471805599453


def _loss_kernel(attx_ref, attg_ref, spap_ref, spat_ref, conp_ref, cont_ref,
                 out_ref, ce_acc, spa_acc, con_acc):
    i = pl.program_id(0)

    @pl.when(i == 0)
    def _init():
        ce_acc[...] = jnp.zeros_like(ce_acc)
        spa_acc[...] = jnp.zeros_like(spa_acc)
        con_acc[...] = jnp.zeros_like(con_acc)

    x0 = attx_ref[0]
    x1 = attx_ref[1]
    x2 = attx_ref[2]
    g = attg_ref[...]
    m = jnp.maximum(jnp.maximum(x0, x1), x2)
    s = jnp.exp(x0 - m) + jnp.exp(x1 - m) + jnp.exp(x2 - m)
    lse = m + jnp.log(s)
    xl = jnp.where(g == 0, x0, jnp.where(g == 1, x1, x2))
    ce_acc[...] += lse - xl

    ps = spap_ref[...]
    qs = jnp.where(spat_ref[...] == 1, ps, 1.0 - ps)
    spa_acc[...] += jnp.log2(jnp.maximum(qs, 1e-7))

    pc = conp_ref[...]
    qc = jnp.where(cont_ref[...] == 1, pc, 1.0 - pc)
    lc = jnp.log2(jnp.maximum(qc, 1e-7))
    con_acc[...] += jnp.sum(lc, axis=0, keepdims=True)

    @pl.when(i == _GRID - 1)
    def _fin():
        att = jnp.sum(ce_acc[...]) * (1.0 / _K)
        spa = jnp.sum(spa_acc[...]) * (-_LN2 / (_K * 6))
        con = jnp.sum(con_acc[...]) * (-_LN2 / (_K * 17))
        out_ref[0] = att
        out_ref[1] = spa
        out_ref[2] = con
        out_ref[3] = att + spa + con


def kernel(att_logits, spa_probs, con_probs, att_gt, spa_gt, con_gt):
    attx = att_logits.T.reshape(3, _K // 128, 128)
    attg = att_gt.astype(jnp.int32).reshape(_K // 128, 128)
    spap = spa_probs.T
    spat = spa_gt.T
    conp = con_probs.T
    cont = con_gt.T

    out = pl.pallas_call(
        _loss_kernel,
        grid=(_GRID,),
        in_specs=[
            pl.BlockSpec((3, _AB, 128), lambda i: (0, i, 0)),
            pl.BlockSpec((_AB, 128), lambda i: (i, 0)),
            pl.BlockSpec((6, _B), lambda i: (0, i)),
            pl.BlockSpec((6, _B), lambda i: (0, i)),
            pl.BlockSpec((17, _B), lambda i: (0, i)),
            pl.BlockSpec((17, _B), lambda i: (0, i)),
        ],
        out_specs=pl.BlockSpec(memory_space=pltpu.MemorySpace.SMEM),
        out_shape=jax.ShapeDtypeStruct((4,), jnp.float32),
        scratch_shapes=[
            pltpu.VMEM((_AB, 128), jnp.float32),
            pltpu.VMEM((6, _B), jnp.float32),
            pltpu.VMEM((1, _B), jnp.float32),
        ],
        compiler_params=pltpu.CompilerParams(
            dimension_semantics=("arbitrary",),
        ),
    )(attx, attg, spap, spat, conp, cont)

    return (out[0], out[1], out[2], out[3])
